# baseline (speedup 1.0000x reference)
import jax
import jax.numpy as jnp
from jax import lax
from jax.experimental import pallas as pl

_NUM_CODES = 1024
_DIM = 256
_ROWS = 8192
_BLK = 512
_COMMIT = 0.25


def _row_sumsq(x):
    sq = x * x
    p = sq[:, :128] + sq[:, 128:]
    acc = p[:, 0:8]
    for k in range(1, 16):
        acc = acc + p[:, 8 * k:8 * (k + 1)]
    t = acc[:, 0:4] + acc[:, 4:8]
    t = t[:, 0:2] + t[:, 2:4]
    return t[:, 0:1] + t[:, 1:2]


def _vq_block(x_ref, e_ref, y2_ref, out_ref, acc_ref):
    i = pl.program_id(0)
    x = x_ref[...]
    e = e_ref[...]
    y2 = y2_ref[...][None, :]
    x2 = _row_sumsq(x)

    mm = lax.dot_general(x.astype(jnp.bfloat16), e.astype(jnp.bfloat16),
                         (((1,), (1,)), ((), ())),
                         preferred_element_type=jnp.float32)
    d2 = (x2 + y2) - 2.0 * mm
    dist = jnp.sqrt(jnp.maximum(d2, 0.0))

    minv = jnp.min(dist, axis=1, keepdims=True)
    ji = lax.broadcasted_iota(jnp.int32, (_BLK, _NUM_CODES), 1)
    idx = jnp.min(jnp.where(dist == minv, ji, _NUM_CODES),
                  axis=1, keepdims=True)
    onehot = (ji == idx).astype(jnp.float32)

    zq = lax.dot_general(onehot, e, (((1,), (0,)), ((), ())),
                         preferred_element_type=jnp.float32,
                         precision=lax.Precision.HIGHEST)
    ste = x + (zq - x)
    out_ref[...] = ste

    part = jnp.sum((ste - x) ** 2)[None, None]

    @pl.when(i == 0)
    def _():
        acc_ref[...] = part

    @pl.when(i != 0)
    def _():
        acc_ref[...] += part


def kernel(z, embedding):
    b, c, l = z.shape
    zp = jnp.transpose(z, (0, 2, 1))
    z_flat = zp.reshape(-1, c)
    y2 = jnp.sum(embedding * embedding, axis=1)

    grid = _ROWS // _BLK
    ste_flat, acc = pl.pallas_call(
        _vq_block,
        grid=(grid,),
        in_specs=[
            pl.BlockSpec((_BLK, _DIM), lambda i: (i, 0)),
            pl.BlockSpec((_NUM_CODES, _DIM), lambda i: (0, 0)),
            pl.BlockSpec((_NUM_CODES,), lambda i: (0,)),
        ],
        out_specs=[
            pl.BlockSpec((_BLK, _DIM), lambda i: (i, 0)),
            pl.BlockSpec((1, 1), lambda i: (0, 0)),
        ],
        out_shape=[
            jax.ShapeDtypeStruct((_ROWS, _DIM), jnp.float32),
            jax.ShapeDtypeStruct((1, 1), jnp.float32),
        ],
    )(z_flat, embedding, y2)

    z_q_out = jnp.transpose(ste_flat.reshape(b, l, c), (0, 2, 1))
    m = acc[0, 0] / jnp.float32(b * c * l)
    qut_loss = m + jnp.float32(_COMMIT) * m
    return (z_q_out, qut_loss)


# speedup vs baseline: 1.2467x; 1.2467x over previous
import jax
import jax.numpy as jnp
from jax import lax
from jax.experimental import pallas as pl

_NUM_CODES = 1024
_DIM = 256
_ROWS = 8192
_BLK = 512
_COMMIT = 0.25


def _row_sumsq(x):
    sq = x * x
    p = sq[:, :128] + sq[:, 128:]
    acc = p[:, 0:8]
    for k in range(1, 16):
        acc = acc + p[:, 8 * k:8 * (k + 1)]
    t = acc[:, 0:4] + acc[:, 4:8]
    t = t[:, 0:2] + t[:, 2:4]
    return t[:, 0:1] + t[:, 1:2]


def _vq_block(x_ref, e_ref, y2_ref, out_ref, acc_ref):
    i = pl.program_id(0)
    x = x_ref[...]
    e = e_ref[...]
    y2 = y2_ref[...][None, :]
    x2 = _row_sumsq(x)

    mm = lax.dot_general(x.astype(jnp.bfloat16), e.astype(jnp.bfloat16),
                         (((1,), (1,)), ((), ())),
                         preferred_element_type=jnp.float32)
    d2 = (x2 + y2) - 2.0 * mm
    dist = jnp.sqrt(jnp.maximum(d2, 0.0))

    minv = jnp.min(dist, axis=1, keepdims=True)
    ji = lax.broadcasted_iota(jnp.int32, (_BLK, _NUM_CODES), 1)
    idx = jnp.min(jnp.where(dist == minv, ji, _NUM_CODES),
                  axis=1, keepdims=True)
    onehot = (ji == idx).astype(jnp.float32)

    zq = lax.dot_general(onehot.astype(jnp.bfloat16), e.astype(jnp.bfloat16),
                         (((1,), (0,)), ((), ())),
                         preferred_element_type=jnp.float32)
    ste = x + (zq - x)
    out_ref[...] = ste

    part = jnp.sum((ste - x) ** 2)[None, None]

    @pl.when(i == 0)
    def _():
        acc_ref[...] = part

    @pl.when(i != 0)
    def _():
        acc_ref[...] += part


def kernel(z, embedding):
    b, c, l = z.shape
    zp = jnp.transpose(z, (0, 2, 1))
    z_flat = zp.reshape(-1, c)
    y2 = jnp.sum(embedding * embedding, axis=1)

    grid = _ROWS // _BLK
    ste_flat, acc = pl.pallas_call(
        _vq_block,
        grid=(grid,),
        in_specs=[
            pl.BlockSpec((_BLK, _DIM), lambda i: (i, 0)),
            pl.BlockSpec((_NUM_CODES, _DIM), lambda i: (0, 0)),
            pl.BlockSpec((_NUM_CODES,), lambda i: (0,)),
        ],
        out_specs=[
            pl.BlockSpec((_BLK, _DIM), lambda i: (i, 0)),
            pl.BlockSpec((1, 1), lambda i: (0, 0)),
        ],
        out_shape=[
            jax.ShapeDtypeStruct((_ROWS, _DIM), jnp.float32),
            jax.ShapeDtypeStruct((1, 1), jnp.float32),
        ],
    )(z_flat, embedding, y2)

    z_q_out = jnp.transpose(ste_flat.reshape(b, l, c), (0, 2, 1))
    m = acc[0, 0] / jnp.float32(b * c * l)
    qut_loss = m + jnp.float32(_COMMIT) * m
    return (z_q_out, qut_loss)


# speedup vs baseline: 3.1798x; 2.5506x over previous
import jax
import jax.numpy as jnp
from jax import lax
from jax.experimental import pallas as pl

_NUM_CODES = 1024
_DIM = 256
_BLK = 512
_COMMIT = 0.25


def _col_sumsq(xt):
    sq = xt * xt
    p = sq[:128, :] + sq[128:, :]
    acc = p[0:8, :]
    for k in range(1, 16):
        acc = acc + p[8 * k:8 * (k + 1), :]
    t = acc[0:4, :] + acc[4:8, :]
    t = t[0:2, :] + t[2:4, :]
    return t[0:1, :] + t[1:2, :]


def _vq_block(x_ref, e_ref, y2_ref, out_ref, acc_ref):
    i = pl.program_id(0)
    xt = x_ref[0]
    e = e_ref[...]
    y2 = y2_ref[...][:, None]
    x2 = _col_sumsq(xt)

    mm = lax.dot_general(e.astype(jnp.bfloat16), xt.astype(jnp.bfloat16),
                         (((1,), (0,)), ((), ())),
                         preferred_element_type=jnp.float32)
    d2 = (x2 + y2) - 2.0 * mm
    dist = jnp.sqrt(jnp.maximum(d2, 0.0))

    minv = jnp.min(dist, axis=0, keepdims=True)
    ji = lax.broadcasted_iota(jnp.int32, (_NUM_CODES, _BLK), 0)
    idx = jnp.min(jnp.where(dist == minv, ji, _NUM_CODES),
                  axis=0, keepdims=True)
    onehot = (ji == idx).astype(jnp.bfloat16)

    zq = lax.dot_general(e.astype(jnp.bfloat16), onehot,
                         (((0,), (0,)), ((), ())),
                         preferred_element_type=jnp.float32)
    ste = xt + (zq - xt)
    out_ref[...] = ste[None]

    part = jnp.sum((ste - xt) ** 2)[None, None]

    @pl.when(i == 0)
    def _():
        acc_ref[...] = part

    @pl.when(i != 0)
    def _():
        acc_ref[...] += part


def kernel(z, embedding):
    b, c, l = z.shape
    y2 = jnp.sum(embedding * embedding, axis=1)

    blocks_per_batch = l // _BLK
    grid = b * blocks_per_batch
    z_q_out, acc = pl.pallas_call(
        _vq_block,
        grid=(grid,),
        in_specs=[
            pl.BlockSpec((1, _DIM, _BLK),
                         lambda i: (i // blocks_per_batch, 0, i % blocks_per_batch)),
            pl.BlockSpec((_NUM_CODES, _DIM), lambda i: (0, 0)),
            pl.BlockSpec((_NUM_CODES,), lambda i: (0,)),
        ],
        out_specs=[
            pl.BlockSpec((1, _DIM, _BLK),
                         lambda i: (i // blocks_per_batch, 0, i % blocks_per_batch)),
            pl.BlockSpec((1, 1), lambda i: (0, 0)),
        ],
        out_shape=[
            jax.ShapeDtypeStruct((b, c, l), jnp.float32),
            jax.ShapeDtypeStruct((1, 1), jnp.float32),
        ],
    )(z, embedding, y2)

    m = acc[0, 0] / jnp.float32(b * c * l)
    qut_loss = m + jnp.float32(_COMMIT) * m
    return (z_q_out, qut_loss)


# speedup vs baseline: 3.6482x; 1.1473x over previous
import jax
import jax.numpy as jnp
from jax import lax
from jax.experimental import pallas as pl

_NUM_CODES = 1024
_DIM = 256
_BLK = 1024
_COMMIT = 0.25


def _col_sumsq(xt):
    sq = xt * xt
    p = sq[:128, :] + sq[128:, :]
    acc = p[0:8, :]
    for k in range(1, 16):
        acc = acc + p[8 * k:8 * (k + 1), :]
    t = acc[0:4, :] + acc[4:8, :]
    t = t[0:2, :] + t[2:4, :]
    return t[0:1, :] + t[1:2, :]


def _vq_block(x_ref, e_ref, y2_ref, out_ref, acc_ref):
    i = pl.program_id(0)
    xt = x_ref[0]
    e = e_ref[...]
    y2 = y2_ref[...][:, None]
    x2 = _col_sumsq(xt)

    em2 = (-2.0 * e).astype(jnp.bfloat16)
    mm2 = lax.dot_general(em2, xt.astype(jnp.bfloat16),
                          (((1,), (0,)), ((), ())),
                          preferred_element_type=jnp.float32)
    d2 = (x2 + y2) + mm2
    dist = jnp.sqrt(jnp.maximum(d2, 0.0))

    minv = jnp.min(dist, axis=0, keepdims=True)
    ji = lax.broadcasted_iota(jnp.int32, (_NUM_CODES, _BLK), 0)
    idx = jnp.min(jnp.where(dist == minv, ji, _NUM_CODES),
                  axis=0, keepdims=True)
    onehot = (ji == idx).astype(jnp.bfloat16)

    zq = lax.dot_general(e.astype(jnp.bfloat16), onehot,
                         (((0,), (0,)), ((), ())),
                         preferred_element_type=jnp.float32)
    delta = zq - xt
    out_ref[...] = (xt + delta)[None]

    part = jnp.sum(delta * delta)[None, None]

    @pl.when(i == 0)
    def _():
        acc_ref[...] = part

    @pl.when(i != 0)
    def _():
        acc_ref[...] += part


def kernel(z, embedding):
    b, c, l = z.shape
    y2 = jnp.sum(embedding * embedding, axis=1)

    blocks_per_batch = max(l // _BLK, 1)
    grid = b * blocks_per_batch
    z_q_out, acc = pl.pallas_call(
        _vq_block,
        grid=(grid,),
        in_specs=[
            pl.BlockSpec((1, _DIM, _BLK),
                         lambda i: (i // blocks_per_batch, 0, i % blocks_per_batch)),
            pl.BlockSpec((_NUM_CODES, _DIM), lambda i: (0, 0)),
            pl.BlockSpec((_NUM_CODES,), lambda i: (0,)),
        ],
        out_specs=[
            pl.BlockSpec((1, _DIM, _BLK),
                         lambda i: (i // blocks_per_batch, 0, i % blocks_per_batch)),
            pl.BlockSpec((1, 1), lambda i: (0, 0)),
        ],
        out_shape=[
            jax.ShapeDtypeStruct((b, c, l), jnp.float32),
            jax.ShapeDtypeStruct((1, 1), jnp.float32),
        ],
    )(z, embedding, y2)

    m = acc[0, 0] / jnp.float32(b * c * l)
    qut_loss = m + jnp.float32(_COMMIT) * m
    return (z_q_out, qut_loss)
